# baseline (speedup 1.0000x reference)
import functools

import jax
import jax.numpy as jnp
from jax import lax
from jax.experimental import pallas as pl
from jax.experimental.pallas import tpu as pltpu
from jax.experimental.pallas import tpu_sc as plsc

EMBED_DIM = 128
_NC = 2
_NS = 16
_NW = _NC * _NS

_ND, _NTM, _NX, _NY, _NU = 76, 49, 202, 202, 48
_YPAD = 208
_DTU = _NTM * _NU


def _ttd_body(wt_ref, wtd_ref, out_ref):
    res = wt_ref[...][:, None, :] + wtd_ref[...][None, :, :]
    out_ref[...] = res.reshape(_DTU, EMBED_DIM)


def _build_ttd(W_time, W_td):
    return pl.pallas_call(
        _ttd_body,
        out_shape=jax.ShapeDtypeStruct((_DTU, EMBED_DIM), jnp.float32),
    )(W_time, W_td)


_DBLK = 4
_XBLK = 101


def _dttd_body(wd_ref, wttd_ref, out_ref):
    wd = wd_ref[pl.ds(pl.program_id(0) * _DBLK, _DBLK), :]
    res = wd[:, None, :] + wttd_ref[...][None, :, :]
    out_ref[...] = res.reshape(_DBLK * _DTU, EMBED_DIM)


def _lxly_body(wx_ref, wy_ref, out_ref):
    wx = wx_ref[pl.ds(pl.program_id(0) * _XBLK, _XBLK), :]
    res = wx[:, None, :] + wy_ref[...][None, :, :]
    res = jnp.concatenate(
        [res, jnp.zeros((_XBLK, _YPAD - _NY, EMBED_DIM), jnp.float32)], axis=1)
    out_ref[...] = res.reshape(_XBLK * _YPAD, EMBED_DIM)


def _build_dttd(W_day, W_time, W_td):
    wttd = _build_ttd(W_time, W_td)
    return pl.pallas_call(
        _dttd_body,
        grid=(_ND // _DBLK,),
        in_specs=[
            pl.BlockSpec((_ND, EMBED_DIM), lambda d: (0, 0)),
            pl.BlockSpec((_DTU, EMBED_DIM), lambda d: (0, 0)),
        ],
        out_specs=pl.BlockSpec((_DBLK * _DTU, EMBED_DIM), lambda d: (d, 0)),
        out_shape=jax.ShapeDtypeStruct((_ND * _DTU, EMBED_DIM), jnp.float32),
    )(W_day, wttd)


def _build_lxly(W_lx, W_ly):
    return pl.pallas_call(
        _lxly_body,
        grid=(_NX // _XBLK,),
        in_specs=[
            pl.BlockSpec((_NX, EMBED_DIM), lambda i: (0, 0)),
            pl.BlockSpec((_NY, EMBED_DIM), lambda i: (0, 0)),
        ],
        out_specs=pl.BlockSpec((_XBLK * _YPAD, EMBED_DIM), lambda i: (i, 0)),
        out_shape=jax.ShapeDtypeStruct((_NX * _YPAD, EMBED_DIM), jnp.float32),
    )(W_lx, W_ly)


def _sc_main(day, time, lx, ly, td, Wdttd, Wlxly, N, C):
    n_w = N // _NW
    n_chunks = n_w // C
    assert n_w % C == 0 and n_chunks % 3 == 2 and n_chunks >= 6
    assert C % 16 == 0
    mesh = plsc.VectorSubcoreMesh(core_axis_name="c", subcore_axis_name="s")

    vmem_sets = []
    for _ in range(3):
        vmem_sets += [
            pltpu.VMEM((5, C), jnp.int32),
            pltpu.VMEM((C,), jnp.int32),
            pltpu.VMEM((C,), jnp.int32),
            pltpu.VMEM((C, EMBED_DIM), jnp.float32),
            pltpu.VMEM((C, EMBED_DIM), jnp.float32),
            pltpu.SemaphoreType.DMA,
            pltpu.SemaphoreType.DMA,
            pltpu.SemaphoreType.DMA,
        ]

    @functools.partial(
        pl.kernel,
        mesh=mesh,
        out_type=jax.ShapeDtypeStruct((N, EMBED_DIM), jnp.float32),
        scratch_types=vmem_sets,
    )
    def k(d_h, t_h, lx_h, ly_h, td_h, wdttd_h, wlxly_h, out_h, *scr):
        sets = [scr[8 * s: 8 * (s + 1)] for s in range(3)]
        idx_hs = (d_h, t_h, lx_h, ly_h, td_h)
        w_hs = (wdttd_h, wlxly_h)
        wid = lax.axis_index("s") * _NC + lax.axis_index("c")
        w_base = wid * n_w

        def fire_idx(c, s):
            ib, gi = sets[s][0], sets[s][5]
            sl = pl.ds(w_base + c * C, C)
            for t in range(5):
                pltpu.async_copy(idx_hs[t].at[sl], ib.at[t], gi)

        def wait_idx(s):
            ib, gi = sets[s][0], sets[s][5]
            sl = pl.ds(0, C)
            for t in range(5):
                pltpu.make_async_copy(idx_hs[t].at[sl], ib.at[t], gi).wait()

        def compute_gidx(s):
            ib, ga, gl = sets[s][0], sets[s][1], sets[s][2]
            for v in range(C // 16):
                dsl = pl.ds(v * 16, 16)
                d = ib[0, dsl]
                t = ib[1, dsl]
                x = ib[2, dsl]
                y = ib[3, dsl]
                u = ib[4, dsl]
                ga[dsl] = (d * _NTM + t) * _NU + u
                gl[dsl] = x * _YPAD + y

        def fire_g(s):
            ga, gl, rb, ob, gg = (sets[s][1], sets[s][2], sets[s][3],
                                  sets[s][4], sets[s][6])
            pltpu.async_copy(w_hs[0].at[ga], ob, gg)
            pltpu.async_copy(w_hs[1].at[gl], rb, gg)

        def wait_g(s):
            ga, gl, rb, ob, gg = (sets[s][1], sets[s][2], sets[s][3],
                                  sets[s][4], sets[s][6])
            pltpu.make_async_copy(w_hs[0].at[ga], ob, gg).wait()
            pltpu.make_async_copy(w_hs[1].at[gl], rb, gg).wait()

        def fire_store(c, s):
            ob, gs = sets[s][4], sets[s][7]
            sl = pl.ds(w_base + c * C, C)
            pltpu.async_copy(ob, out_h.at[sl], gs)

        def wait_store(s):
            ob, gs = sets[s][4], sets[s][7]
            sl = pl.ds(0, C)
            pltpu.make_async_copy(ob, out_h.at[sl], gs).wait()

        def acc(s):
            rb, ob = sets[s][3], sets[s][4]

            def body(p, carry):
                for j in range(EMBED_DIM // 16):
                    dsl = pl.ds(j * 16, 16)
                    plsc.addupdate(ob.at[p, dsl], rb[p, dsl])
                return carry

            lax.fori_loop(0, C, body, 0)

        def step(c, s, do_wait_store, guard_idx):
            s2 = (s + 2) % 3
            wait_g(s)
            wait_idx(s2)
            compute_gidx(s2)
            if do_wait_store:
                wait_store(s2)
            fire_g(s2)
            if guard_idx:
                @pl.when(c + 3 < n_chunks)
                def _():
                    fire_idx(c + 3, s)
            else:
                fire_idx(c + 3, s)
            acc(s)
            fire_store(c, s)

        for s in range(3):
            fire_idx(s, s)
        for s in range(2):
            wait_idx(s)
            compute_gidx(s)
            fire_g(s)

        for c in range(3):
            step(c, c, do_wait_store=(c > 0), guard_idx=False)

        def triple(k_, carry):
            c0 = 3 * k_
            step(c0, 0, do_wait_store=True, guard_idx=True)
            step(c0 + 1, 1, do_wait_store=True, guard_idx=True)
            step(c0 + 2, 2, do_wait_store=True, guard_idx=True)
            return carry

        lax.fori_loop(1, (n_chunks - 2) // 3, triple, 0)

        for c in (n_chunks - 2, n_chunks - 1):
            s = c % 3
            wait_g(s)
            acc(s)
            fire_store(c, s)

        for s in range(3):
            wait_store(s)

    return k(day, time, lx, ly, td, Wdttd, Wlxly)


@functools.partial(jax.jit, static_argnums=(10, 11))
def _lookup_sum(day, time, lx, ly, td, W_day, W_time, W_lx, W_ly, W_td, N, C):
    Wdttd = _build_dttd(W_day, W_time, W_td)
    Wlxly = _build_lxly(W_lx, W_ly)
    return _sc_main(day, time, lx, ly, td, Wdttd, Wlxly, N, C)


def kernel(day, time, location_x, location_y, timedelta,
           W_day, W_time, W_lx, W_ly, W_td):
    B, L = day.shape
    N = B * L

    def flat(a):
        return a.reshape(-1).astype(jnp.int32)

    out = _lookup_sum(flat(day), flat(time), flat(location_x),
                      flat(location_y), flat(timedelta),
                      W_day, W_time, W_lx, W_ly, W_td, N, 80)
    return out.reshape(B, L, EMBED_DIM)


# speedup vs baseline: 1.0062x; 1.0062x over previous
import functools

import jax
import jax.numpy as jnp
from jax import lax
from jax.experimental import pallas as pl
from jax.experimental.pallas import tpu as pltpu
from jax.experimental.pallas import tpu_sc as plsc

EMBED_DIM = 128
_NC = 2
_NS = 16
_NW = _NC * _NS

_ND, _NTM, _NX, _NY, _NU = 76, 49, 202, 202, 48
_YPAD = 208
_DTU = _NTM * _NU


_DBLK = 4
_XBLK = 101


def _dttd_body(wd_ref, wt_ref, wu_ref, out_ref):
    wtu = wt_ref[...][:, None, :] + wu_ref[...][None, :, :]
    wttd = wtu.reshape(_DTU, EMBED_DIM)
    wd = wd_ref[pl.ds(pl.program_id(0) * _DBLK, _DBLK), :]
    res = wd[:, None, :] + wttd[None, :, :]
    out_ref[...] = res.reshape(_DBLK * _DTU, EMBED_DIM)


def _lxly_body(wx_ref, wy_ref, out_ref):
    wx = wx_ref[pl.ds(pl.program_id(0) * _XBLK, _XBLK), :]
    res = wx[:, None, :] + wy_ref[...][None, :, :]
    res = jnp.concatenate(
        [res, jnp.zeros((_XBLK, _YPAD - _NY, EMBED_DIM), jnp.float32)], axis=1)
    out_ref[...] = res.reshape(_XBLK * _YPAD, EMBED_DIM)


def _build_dttd(W_day, W_time, W_td):
    return pl.pallas_call(
        _dttd_body,
        grid=(_ND // _DBLK,),
        in_specs=[
            pl.BlockSpec((_ND, EMBED_DIM), lambda d: (0, 0)),
            pl.BlockSpec((_NTM, EMBED_DIM), lambda d: (0, 0)),
            pl.BlockSpec((_NU, EMBED_DIM), lambda d: (0, 0)),
        ],
        out_specs=pl.BlockSpec((_DBLK * _DTU, EMBED_DIM), lambda d: (d, 0)),
        out_shape=jax.ShapeDtypeStruct((_ND * _DTU, EMBED_DIM), jnp.float32),
    )(W_day, W_time, W_td)


def _build_lxly(W_lx, W_ly):
    return pl.pallas_call(
        _lxly_body,
        grid=(_NX // _XBLK,),
        in_specs=[
            pl.BlockSpec((_NX, EMBED_DIM), lambda i: (0, 0)),
            pl.BlockSpec((_NY, EMBED_DIM), lambda i: (0, 0)),
        ],
        out_specs=pl.BlockSpec((_XBLK * _YPAD, EMBED_DIM), lambda i: (i, 0)),
        out_shape=jax.ShapeDtypeStruct((_NX * _YPAD, EMBED_DIM), jnp.float32),
    )(W_lx, W_ly)


def _sc_main(day, time, lx, ly, td, Wdttd, Wlxly, N, C):
    n_w = N // _NW
    n_chunks = n_w // C
    assert n_w % C == 0 and n_chunks % 3 == 2 and n_chunks >= 6
    assert C % 16 == 0
    mesh = plsc.VectorSubcoreMesh(core_axis_name="c", subcore_axis_name="s")

    vmem_sets = []
    for _ in range(3):
        vmem_sets += [
            pltpu.VMEM((5, C), jnp.int32),
            pltpu.VMEM((C,), jnp.int32),
            pltpu.VMEM((C,), jnp.int32),
            pltpu.VMEM((2, C, EMBED_DIM), jnp.float32),
            pltpu.VMEM((C, EMBED_DIM), jnp.float32),
            pltpu.SemaphoreType.DMA,
            pltpu.SemaphoreType.DMA,
            pltpu.SemaphoreType.DMA,
        ]

    @functools.partial(
        pl.kernel,
        mesh=mesh,
        out_type=jax.ShapeDtypeStruct((N, EMBED_DIM), jnp.float32),
        scratch_types=vmem_sets,
    )
    def k(d_h, t_h, lx_h, ly_h, td_h, wdttd_h, wlxly_h, out_h, *scr):
        sets = [scr[8 * s: 8 * (s + 1)] for s in range(3)]
        idx_hs = (d_h, t_h, lx_h, ly_h, td_h)
        w_hs = (wdttd_h, wlxly_h)
        wid = lax.axis_index("s") * _NC + lax.axis_index("c")
        w_base = wid * n_w

        def fire_idx(c, s):
            ib, gi = sets[s][0], sets[s][5]
            sl = pl.ds(w_base + c * C, C)
            for t in range(5):
                pltpu.async_copy(idx_hs[t].at[sl], ib.at[t], gi)

        def wait_idx(s):
            ib, gi = sets[s][0], sets[s][5]
            sl = pl.ds(0, C)
            for t in range(5):
                pltpu.make_async_copy(idx_hs[t].at[sl], ib.at[t], gi).wait()

        def compute_gidx(s):
            ib, ga, gl = sets[s][0], sets[s][1], sets[s][2]
            for v in range(C // 16):
                dsl = pl.ds(v * 16, 16)
                d = ib[0, dsl]
                t = ib[1, dsl]
                x = ib[2, dsl]
                y = ib[3, dsl]
                u = ib[4, dsl]
                ga[dsl] = (d * _NTM + t) * _NU + u
                gl[dsl] = x * _YPAD + y

        def fire_g(s):
            ga, gl, rb, gg = sets[s][1], sets[s][2], sets[s][3], sets[s][6]
            pltpu.async_copy(w_hs[0].at[ga], rb.at[0], gg)
            pltpu.async_copy(w_hs[1].at[gl], rb.at[1], gg)

        def wait_g(s):
            ga, gl, rb, gg = sets[s][1], sets[s][2], sets[s][3], sets[s][6]
            pltpu.make_async_copy(w_hs[0].at[ga], rb.at[0], gg).wait()
            pltpu.make_async_copy(w_hs[1].at[gl], rb.at[1], gg).wait()

        def fire_store(c, s):
            ob, gs = sets[s][4], sets[s][7]
            sl = pl.ds(w_base + c * C, C)
            pltpu.async_copy(ob, out_h.at[sl], gs)

        def wait_store(s):
            ob, gs = sets[s][4], sets[s][7]
            sl = pl.ds(0, C)
            pltpu.make_async_copy(ob, out_h.at[sl], gs).wait()

        def acc(s):
            rb, ob = sets[s][3], sets[s][4]

            def body(p, carry):
                for j in range(EMBED_DIM // 16):
                    dsl = pl.ds(j * 16, 16)
                    ob[p, dsl] = rb[0, p, dsl] + rb[1, p, dsl]
                return carry

            lax.fori_loop(0, C, body, 0)

        def step(c, s, do_wait_store, guard_idx):
            s2 = (s + 2) % 3
            wait_g(s)
            wait_idx(s2)
            compute_gidx(s2)
            fire_g(s2)
            if guard_idx:
                @pl.when(c + 3 < n_chunks)
                def _():
                    fire_idx(c + 3, s)
            else:
                fire_idx(c + 3, s)
            if do_wait_store:
                wait_store(s)
            acc(s)
            fire_store(c, s)

        for s in range(3):
            fire_idx(s, s)
        for s in range(2):
            wait_idx(s)
            compute_gidx(s)
            fire_g(s)

        for c in range(3):
            step(c, c, do_wait_store=False, guard_idx=False)

        def triple(k_, carry):
            c0 = 3 * k_
            step(c0, 0, do_wait_store=True, guard_idx=True)
            step(c0 + 1, 1, do_wait_store=True, guard_idx=True)
            step(c0 + 2, 2, do_wait_store=True, guard_idx=True)
            return carry

        lax.fori_loop(1, (n_chunks - 2) // 3, triple, 0)

        for c in (n_chunks - 2, n_chunks - 1):
            s = c % 3
            wait_g(s)
            wait_store(s)
            acc(s)
            fire_store(c, s)

        for s in range(3):
            wait_store(s)

    return k(day, time, lx, ly, td, Wdttd, Wlxly)


@functools.partial(jax.jit, static_argnums=(10, 11))
def _lookup_sum(day, time, lx, ly, td, W_day, W_time, W_lx, W_ly, W_td, N, C):
    Wdttd = _build_dttd(W_day, W_time, W_td)
    Wlxly = _build_lxly(W_lx, W_ly)
    return _sc_main(day, time, lx, ly, td, Wdttd, Wlxly, N, C)


def kernel(day, time, location_x, location_y, timedelta,
           W_day, W_time, W_lx, W_ly, W_td):
    B, L = day.shape
    N = B * L

    def flat(a):
        return a.reshape(-1).astype(jnp.int32)

    out = _lookup_sum(flat(day), flat(time), flat(location_x),
                      flat(location_y), flat(timedelta),
                      W_day, W_time, W_lx, W_ly, W_td, N, 80)
    return out.reshape(B, L, EMBED_DIM)
